# baseline (speedup 1.0000x reference)
import functools

import jax
import jax.numpy as jnp
from jax import lax
from jax.experimental import pallas as pl
from jax.experimental.pallas import tpu as pltpu
from jax.experimental.pallas import tpu_sc as plsc

BATCH = 16384
EMBED_DIM = 64
NROWS = 1000000
PANEL = 256
NPANEL = (NROWS - EMBED_DIM) // PANEL
TAIL0 = NPANEL * PANEL

_info = plsc.get_sparse_core_info()
_NC, _NS, _L = _info.num_cores, _info.num_subcores, _info.num_lanes
_NW = _NC * _NS
_PPW = (NPANEL + _NW - 1) // _NW
_BPW = BATCH // _NW
_CHUNK = 256
_NCHUNK = _BPW // _CHUNK
_GPC = _CHUNK // _L


def _tr_body(utT, itT, o_u, o_i, pu, pi, ou, oi, sem_in, sem_out):
    wid = lax.axis_index("s") * _NC + lax.axis_index("c")
    start = wid * _PPW
    nc = jnp.clip(NPANEL - start, 0, _PPW)

    lanes = lax.iota(jnp.int32, _L)
    half_lanes = lax.shift_right_logical(lanes, 1)
    parity64 = jnp.bitwise_and(lanes, 1) * EMBED_DIM

    def fire_in(c):
        pltpu.async_copy(utT.at[:, pl.ds(c * PANEL, PANEL)], pu, sem_in)
        pltpu.async_copy(itT.at[:, pl.ds(c * PANEL, PANEL)], pi, sem_in)

    def drain_in():
        pltpu.make_async_copy(utT.at[:, pl.ds(0, PANEL)], pu, sem_in).wait()
        pltpu.make_async_copy(itT.at[:, pl.ds(0, PANEL)], pi, sem_in).wait()

    def drain_out(c):
        pltpu.make_async_copy(
            ou, o_u.at[pl.ds(c * (PANEL // 2), PANEL // 2), :],
            sem_out).wait()
        pltpu.make_async_copy(
            oi, o_i.at[pl.ds(c * (PANEL // 2), PANEL // 2), :],
            sem_out).wait()

    @pl.when(nc > 0)
    def _():
        fire_in(start)

    def panel_body(t, _):
        c = start + t
        drain_in()

        @pl.when(t > 0)
        def _():
            drain_out(c - 1)

        def d_body(d, _):
            col16 = parity64 + d
            for g in range(PANEL // _L):
                row16 = g * (_L // 2) + half_lanes
                plsc.store_scatter(ou, [row16, col16],
                                   pu[d, pl.ds(g * _L, _L)])
                plsc.store_scatter(oi, [row16, col16],
                                   pi[d, pl.ds(g * _L, _L)])
            return 0
        lax.fori_loop(0, EMBED_DIM, d_body, 0, unroll=4)

        pltpu.async_copy(ou, o_u.at[pl.ds(c * (PANEL // 2), PANEL // 2), :],
                         sem_out)
        pltpu.async_copy(oi, o_i.at[pl.ds(c * (PANEL // 2), PANEL // 2), :],
                         sem_out)

        @pl.when(t + 1 < nc)
        def _():
            fire_in(c + 1)
        return 0

    lax.fori_loop(0, nc, panel_body, 0)

    @pl.when(nc > 0)
    def _():
        drain_out(start + nc - 1)


def _mf_body(user_id_hbm, item_id_hbm, ut_hbm, it_hbm, w_hbm,
             tu_hbm, ti_hbm, out_hbm, ids_u, ids_i, idx_u, idx_i,
             g_u, g_i, w_v, tu_v, ti_v, out_v, sem):
    wid = lax.axis_index("s") * _NC + lax.axis_index("c")
    base = wid * _BPW

    pltpu.sync_copy(user_id_hbm.at[pl.ds(base, _BPW)], ids_u)
    pltpu.sync_copy(item_id_hbm.at[pl.ds(base, _BPW)], ids_i)
    pltpu.sync_copy(w_hbm, w_v)
    pltpu.sync_copy(tu_hbm, tu_v)
    pltpu.sync_copy(ti_hbm, ti_v)

    lanes = lax.iota(jnp.int32, _L)
    maxid = jnp.full((_L,), TAIL0 - 1, jnp.int32)

    def chunk_body(k, _):
        def ridx_body(g, _):
            u16 = ids_u[pl.ds(k * _CHUNK + g * _L, _L)]
            i16 = ids_i[pl.ds(k * _CHUNK + g * _L, _L)]
            idx_u[pl.ds(g * _L, _L)] = lax.shift_right_logical(
                jnp.minimum(u16, maxid), 1)
            idx_i[pl.ds(g * _L, _L)] = lax.shift_right_logical(
                jnp.minimum(i16, maxid), 1)
            return 0
        lax.fori_loop(0, _GPC, ridx_body, 0)

        cu = pltpu.async_copy(ut_hbm.at[idx_u], g_u, sem)
        ci = pltpu.async_copy(it_hbm.at[idx_i], g_i, sem)
        cu.wait()
        ci.wait()

        def group_body(g, _):
            u16 = ids_u[pl.ds(k * _CHUNK + g * _L, _L)]
            i16 = ids_i[pl.ds(k * _CHUNK + g * _L, _L)]
            uoff = jnp.bitwise_and(u16, 1) * EMBED_DIM
            ioff = jnp.bitwise_and(i16, 1) * EMBED_DIM
            jj = g * _L + lanes

            def d_body(d, acc):
                dd = jnp.full((_L,), d, jnp.int32)
                ug = plsc.load_gather(g_u, [jj, uoff + dd])
                ig = plsc.load_gather(g_i, [jj, ioff + dd])
                wg = plsc.load_gather(w_v, [dd])
                return acc + ug * ig * wg

            acc = lax.fori_loop(0, EMBED_DIM, d_body,
                                jnp.zeros((_L,), jnp.float32), unroll=8)

            tmask_u = u16 >= TAIL0
            tmask_i = i16 >= TAIL0
            ntail = plsc.all_reduce_population_count(
                jnp.logical_or(tmask_u, tmask_i))

            @pl.when(ntail[0] > 0)
            def _():
                ut16 = jnp.maximum(u16 - TAIL0, 0)
                it16 = jnp.maximum(i16 - TAIL0, 0)

                def dt_body(d, acc2):
                    dd = jnp.full((_L,), d, jnp.int32)
                    ug = plsc.load_gather(g_u, [jj, uoff + dd])
                    ig = plsc.load_gather(g_i, [jj, ioff + dd])
                    tug = plsc.load_gather(tu_v, [ut16, dd])
                    tig = plsc.load_gather(ti_v, [it16, dd])
                    ugf = jnp.where(tmask_u, tug, ug)
                    igf = jnp.where(tmask_i, tig, ig)
                    wg = plsc.load_gather(w_v, [dd])
                    return acc2 + ugf * igf * wg

                acc2 = lax.fori_loop(0, EMBED_DIM, dt_body,
                                     jnp.zeros((_L,), jnp.float32))
                out_v[pl.ds(k * _CHUNK + g * _L, _L)] = acc2

            @pl.when(ntail[0] == 0)
            def _():
                out_v[pl.ds(k * _CHUNK + g * _L, _L)] = acc
            return 0
        lax.fori_loop(0, _GPC, group_body, 0)
        return 0

    lax.fori_loop(0, _NCHUNK, chunk_body, 0)

    pltpu.sync_copy(out_v, out_hbm.at[pl.ds(base, _BPW)])


@jax.jit
def _gmf(user_id, item_id, user_table, item_table, predict_w):
    mesh = plsc.VectorSubcoreMesh(core_axis_name="c", subcore_axis_name="s")
    cp = pltpu.CompilerParams(needs_layout_passes=False)

    o_u, o_i = pl.kernel(
        _tr_body,
        mesh=mesh,
        compiler_params=cp,
        out_type=(jax.ShapeDtypeStruct((NROWS // 2, 2 * EMBED_DIM),
                                       jnp.float32),
                  jax.ShapeDtypeStruct((NROWS // 2, 2 * EMBED_DIM),
                                       jnp.float32)),
        scratch_types=[
            pltpu.VMEM((EMBED_DIM, PANEL), jnp.float32),
            pltpu.VMEM((EMBED_DIM, PANEL), jnp.float32),
            pltpu.VMEM((PANEL // 2, 2 * EMBED_DIM), jnp.float32),
            pltpu.VMEM((PANEL // 2, 2 * EMBED_DIM), jnp.float32),
            pltpu.SemaphoreType.DMA,
            pltpu.SemaphoreType.DMA,
        ],
    )(user_table.T, item_table.T)

    tails_u = jnp.pad(user_table[TAIL0:], ((0, 0), (0, EMBED_DIM)))
    tails_i = jnp.pad(item_table[TAIL0:], ((0, 0), (0, EMBED_DIM)))

    return pl.kernel(
        _mf_body,
        mesh=mesh,
        compiler_params=cp,
        out_type=jax.ShapeDtypeStruct((BATCH,), jnp.float32),
        scratch_types=[
            pltpu.VMEM((_BPW,), jnp.int32),
            pltpu.VMEM((_BPW,), jnp.int32),
            pltpu.VMEM((_CHUNK,), jnp.int32),
            pltpu.VMEM((_CHUNK,), jnp.int32),
            pltpu.VMEM((_CHUNK, 2 * EMBED_DIM), jnp.float32),
            pltpu.VMEM((_CHUNK, 2 * EMBED_DIM), jnp.float32),
            pltpu.VMEM((EMBED_DIM,), jnp.float32),
            pltpu.VMEM((EMBED_DIM, 2 * EMBED_DIM), jnp.float32),
            pltpu.VMEM((EMBED_DIM, 2 * EMBED_DIM), jnp.float32),
            pltpu.VMEM((_BPW,), jnp.float32),
            pltpu.SemaphoreType.DMA,
        ],
    )(user_id, item_id, o_u, o_i, predict_w, tails_u, tails_i)


def kernel(user_id, item_id, user_table, item_table, predict_w):
    return _gmf(user_id.astype(jnp.int32), item_id.astype(jnp.int32),
                user_table, item_table, predict_w)
